# baseline (speedup 1.0000x reference)
import jax
import jax.numpy as jnp
from jax import lax
from jax.experimental import pallas as pl
from jax.experimental.pallas import tpu as pltpu
from jax.experimental.pallas import tpu_sc as plsc

VOCAB_P1 = 100001
BATCH = 4096
HIST = 200
LANES = 16
NUM_CORES = 2
NUM_SUBCORES = 16
NUM_TILES = NUM_CORES * NUM_SUBCORES
ROWS_PER_TILE = BATCH // NUM_TILES
FULL_CHUNKS = HIST // LANES
TAIL_OFF = HIST - LANES
HALF_ROWS = ROWS_PER_TILE // 2
UNROLL = 4


def _sc_body(table_hbm, x_hbm, out_hbm, table_sh, table_v, idx_v, out_v,
             sem_i):
    sid = lax.axis_index("s")
    wid = sid * NUM_CORES + lax.axis_index("c")
    rbase = wid * ROWS_PER_TILE

    cp_i = pltpu.async_copy(x_hbm.at[pl.ds(rbase, HALF_ROWS), :], idx_v, sem_i)

    @pl.when(sid == 0)
    def _():
        pltpu.sync_copy(table_hbm, table_sh)

    plsc.subcore_barrier()
    pltpu.sync_copy(table_sh, table_v)

    lane = lax.iota(jnp.int32, LANES)
    tail_mask = lane >= (LANES - (HIST - FULL_CHUNKS * LANES))
    last_lane = lane == (LANES - 1)

    def make_group_body(out_base):
        def group_body(g, _):
            r0 = g * UNROLL
            accs = [jnp.zeros((LANES,), jnp.float32) for _ in range(UNROLL)]
            for j in range(FULL_CHUNKS):
                for u in range(UNROLL):
                    idx = idx_v[r0 + u, pl.ds(j * LANES, LANES)]
                    accs[u] = accs[u] + plsc.load_gather(table_v, [idx])
            for u in range(UNROLL):
                idx_t = idx_v[r0 + u, pl.ds(TAIL_OFF, LANES)]
                vt = plsc.load_gather(table_v, [idx_t])
                accs[u] = accs[u] + jnp.where(tail_mask, vt, 0.0)
            for u in range(UNROLL):
                total = plsc.cumsum(accs[u])
                plsc.store_scatter(
                    out_v, [jnp.full((LANES,), out_base + r0 + u, jnp.int32)],
                    total, mask=last_lane)
            return ()
        return group_body

    cp_i.wait()
    lax.fori_loop(0, HALF_ROWS // UNROLL, make_group_body(0), ())
    pltpu.async_copy(
        x_hbm.at[pl.ds(rbase + HALF_ROWS, HALF_ROWS), :], idx_v, sem_i).wait()
    lax.fori_loop(0, HALF_ROWS // UNROLL, make_group_body(HALF_ROWS), ())

    pltpu.sync_copy(out_v, out_hbm.at[pl.ds(rbase, ROWS_PER_TILE)])


@jax.jit
def _bow_sum(table_flat, x):
    mesh = plsc.VectorSubcoreMesh(core_axis_name="c", subcore_axis_name="s")
    return pl.kernel(
        _sc_body,
        out_type=jax.ShapeDtypeStruct((BATCH,), jnp.float32),
        mesh=mesh,
        scratch_types=[
            pltpu.VMEM_SHARED((VOCAB_P1,), jnp.float32),
            pltpu.VMEM((VOCAB_P1,), jnp.float32),
            pltpu.VMEM((HALF_ROWS, HIST), jnp.int32),
            pltpu.VMEM((ROWS_PER_TILE,), jnp.float32),
            pltpu.SemaphoreType.DMA,
        ],
        compiler_params=pltpu.CompilerParams(needs_layout_passes=False, skip_device_barrier=True),
    )(table_flat, x)


def kernel(x, table):
    return _bow_sum(table[:, 0], x).reshape(BATCH, 1)


# speedup vs baseline: 1.1832x; 1.1832x over previous
import jax
import jax.numpy as jnp
from jax import lax
from jax.experimental import pallas as pl
from jax.experimental.pallas import tpu as pltpu
from jax.experimental.pallas import tpu_sc as plsc

VOCAB_P1 = 100001
BATCH = 4096
HIST = 200
LANES = 16
NUM_CORES = 2
NUM_SUBCORES = 16
NUM_TILES = NUM_CORES * NUM_SUBCORES
COLS_PER_TILE = BATCH // NUM_TILES
GROUPS = COLS_PER_TILE // LANES
L_UNROLL = 2
L_PASS1 = 104
L_PASS2 = HIST - L_PASS1


def _sc_body(table_hbm, xt_hbm, out_hbm, table_sh, table_v, idx_v, out_v,
             sem_i):
    sid = lax.axis_index("s")
    wid = sid * NUM_CORES + lax.axis_index("c")
    cbase = wid * COLS_PER_TILE

    cp_i = pltpu.async_copy(
        xt_hbm.at[pl.ds(0, L_PASS1), pl.ds(cbase, COLS_PER_TILE)], idx_v,
        sem_i)

    @pl.when(sid == 0)
    def _():
        pltpu.sync_copy(table_hbm, table_sh)

    plsc.subcore_barrier()
    pltpu.sync_copy(table_sh, table_v)

    def step(l0, accs):
        accs = list(accs)
        for dl in range(L_UNROLL):
            l = l0 * L_UNROLL + dl
            for g in range(GROUPS):
                idx = idx_v[l, pl.ds(g * LANES, LANES)]
                accs[g] = accs[g] + plsc.load_gather(table_v, [idx])
        return tuple(accs)

    zero = jnp.zeros((LANES,), jnp.float32)
    cp_i.wait()
    accs = lax.fori_loop(0, L_PASS1 // L_UNROLL, step, (zero,) * GROUPS)
    pltpu.async_copy(
        xt_hbm.at[pl.ds(L_PASS1, L_PASS2), pl.ds(cbase, COLS_PER_TILE)],
        idx_v.at[pl.ds(0, L_PASS2), :], sem_i).wait()
    accs = lax.fori_loop(0, L_PASS2 // L_UNROLL, step, accs)
    for g in range(GROUPS):
        out_v[pl.ds(g * LANES, LANES)] = accs[g]

    pltpu.sync_copy(out_v, out_hbm.at[pl.ds(cbase, COLS_PER_TILE)])


@jax.jit
def _bow_sum(table_flat, xt):
    mesh = plsc.VectorSubcoreMesh(core_axis_name="c", subcore_axis_name="s")
    return pl.kernel(
        _sc_body,
        out_type=jax.ShapeDtypeStruct((BATCH,), jnp.float32),
        mesh=mesh,
        scratch_types=[
            pltpu.VMEM_SHARED((VOCAB_P1,), jnp.float32),
            pltpu.VMEM((VOCAB_P1,), jnp.float32),
            pltpu.VMEM((L_PASS1, COLS_PER_TILE), jnp.int32),
            pltpu.VMEM((COLS_PER_TILE,), jnp.float32),
            pltpu.SemaphoreType.DMA,
        ],
        compiler_params=pltpu.CompilerParams(needs_layout_passes=False),
    )(table_flat, xt)


def kernel(x, table):
    return _bow_sum(table.reshape(-1), x.T).reshape(BATCH, 1)
